# baseline (speedup 1.0000x reference)
import functools

import jax
import jax.numpy as jnp
from jax import lax
from jax.experimental import pallas as pl
from jax.experimental.pallas import tpu as pltpu
from jax.experimental.pallas import tpu_sc as plsc

NC, NS, LANES = 2, 16, 16
NW = NC * NS
C = 128
NBUF = 4
VP = 1024


def _make_sc_kernel(G, D, N):
  mesh = plsc.VectorSubcoreMesh(core_axis_name="c", subcore_axis_name="s")
  grp = D // LANES
  RP = VP // NS

  @functools.partial(
      pl.kernel,
      out_type=jax.ShapeDtypeStruct((N, D), jnp.float32),
      mesh=mesh,
      scratch_types=dict(
          idx_a=pltpu.VMEM((G, C), jnp.int32),
          idx_b=pltpu.VMEM((G, C), jnp.int32),
          idx_c=pltpu.VMEM((G, C), jnp.int32),
          cont_v=pltpu.VMEM((G, C), jnp.float32),
          wv=pltpu.VMEM((D,), jnp.float32),
          bv=pltpu.VMEM((D,), jnp.float32),
          buf_d=pltpu.VMEM((NBUF, C, D), jnp.float32),
          sh_b=pltpu.VMEM_SHARED((VP, D), jnp.float32),
          sh_c=pltpu.VMEM_SHARED((VP, D), jnp.float32),
          sem_a=pltpu.SemaphoreType.DMA((NBUF,)),
          sem_b=pltpu.SemaphoreType.DMA((NBUF,)),
          sem_c=pltpu.SemaphoreType.DMA((NBUF,)),
          sem_w=pltpu.SemaphoreType.DMA((NBUF,)),
      ),
  )
  def sc_kernel(ia_hbm, ib_hbm, ic_hbm, cd_hbm, ea_hbm, eb_hbm, ec_hbm,
                w_hbm, bias_hbm, out_hbm, *, idx_a, idx_b, idx_c, cont_v,
                wv, bv, buf_d, sh_b, sh_c, sem_a, sem_b, sem_c, sem_w):
    sid = lax.axis_index("s")
    wid = sid * NC + lax.axis_index("c")
    base = wid * (G * C)

    pltpu.sync_copy(eb_hbm.at[pl.ds(sid * RP, RP)],
                    sh_b.at[pl.ds(sid * RP, RP)])
    pltpu.sync_copy(ec_hbm.at[pl.ds(sid * RP, RP)],
                    sh_c.at[pl.ds(sid * RP, RP)])

    pltpu.sync_copy(ia_hbm.at[wid], idx_a)
    pltpu.sync_copy(ib_hbm.at[wid], idx_b)
    pltpu.sync_copy(ic_hbm.at[wid], idx_c)
    pltpu.sync_copy(cd_hbm.at[wid], cont_v)
    pltpu.sync_copy(w_hbm, wv)
    pltpu.sync_copy(bias_hbm, bv)

    plsc.subcore_barrier()

    tabs = (ea_hbm, sh_b, sh_c)
    idxs = (idx_a, idx_b, idx_c)
    gsems = (sem_a, sem_b, sem_c)

    wk = [wv[pl.ds(k * LANES, LANES)] for k in range(grp)]
    bk = [bv[pl.ds(k * LANES, LANES)] for k in range(grp)]

    def g_copies(g, s):
      return [pltpu.make_async_copy(tabs[t].at[idxs[t].at[g]],
                                    buf_d.at[s], gsems[t].at[s])
              for t in range(3)]

    def w_copy(g, s):
      return pltpu.make_async_copy(
          buf_d.at[s], out_hbm.at[pl.ds(base + g * C, C)], sem_w.at[s])

    def compute_dense(g, s):
      def row16(j, carry2):
        cv = cont_v[g, pl.ds(j * LANES, LANES)]
        for r in range(LANES):
          c0 = cv[r]
          i = j * LANES + r
          for k in range(grp):
            buf_d[s, i, pl.ds(k * LANES, LANES)] = c0 * wk[k] + bk[k]
        return carry2

      lax.fori_loop(0, C // LANES, row16, 0, unroll=False)

    def body(g, carry):
      s = g % NBUF

      @pl.when(g >= NBUF)
      def _():
        w_copy(g - NBUF, s).wait()

      compute_dense(g, s)
      for t in range(3):
        pltpu.async_copy(tabs[t].at[idxs[t].at[g]], buf_d.at[s],
                         gsems[t].at[s], add=True)

      @pl.when(g >= 2)
      def _():
        sm2 = (g - 2) % NBUF
        for cp in g_copies(g - 2, sm2):
          cp.wait()
        w_copy(g - 2, sm2).start()

      return carry

    lax.fori_loop(0, G, body, 0, unroll=False)

    for gg in (G - 2, G - 1):
      sg = gg % NBUF
      for cp in g_copies(gg, sg):
        cp.wait()
      w_copy(gg, sg).start()
    for gg in range(G - NBUF, G):
      w_copy(gg, gg % NBUF).wait()

  return sc_kernel


def kernel(cat_a, cat_b, cat_c, cont_d, emb_a, emb_b, emb_c, W, b):
  B, S = cat_a.shape
  D = emb_a.shape[1]
  N = B * S
  per_w = N // NW
  G = per_w // C
  ia = cat_a.reshape(NW, G, C)
  ib = cat_b.reshape(NW, G, C)
  ic = cat_c.reshape(NW, G, C)
  cd = cont_d.reshape(NW, G, C)
  ebp = jnp.pad(emb_b, ((0, VP - emb_b.shape[0]), (0, 0)))
  ecp = jnp.pad(emb_c, ((0, VP - emb_c.shape[0]), (0, 0)))
  out = _make_sc_kernel(G, D, N)(
      ia, ib, ic, cd, emb_a, ebp, ecp, W.reshape(D), b)
  return out.reshape(B, S, D)


# speedup vs baseline: 1.0292x; 1.0292x over previous
import functools

import jax
import jax.numpy as jnp
from jax import lax
from jax.experimental import pallas as pl
from jax.experimental.pallas import tpu as pltpu
from jax.experimental.pallas import tpu_sc as plsc

NC, NS, LANES = 2, 16, 16
NW = NC * NS
C = 128
NBUF = 4
V2 = 1000


def _make_sc_kernel(G, D, N):
  mesh = plsc.VectorSubcoreMesh(core_axis_name="c", subcore_axis_name="s")
  grp = D // LANES
  oct_q, oct_r = divmod(V2 // 8, NS)
  R0, R1 = 8 * (oct_q + 1), 8 * oct_q
  CUT = oct_r

  @functools.partial(
      pl.kernel,
      out_type=jax.ShapeDtypeStruct((N, D), jnp.float32),
      mesh=mesh,
      scratch_types=dict(
          idx_a=pltpu.VMEM((G, C), jnp.int32),
          idx_b=pltpu.VMEM((G, C), jnp.int32),
          idx_c=pltpu.VMEM((G, C), jnp.int32),
          cont_v=pltpu.VMEM((G, C), jnp.float32),
          wv=pltpu.VMEM((D,), jnp.float32),
          bv=pltpu.VMEM((D,), jnp.float32),
          buf_d=pltpu.VMEM((NBUF, C, D), jnp.float32),
          sh_b=pltpu.VMEM_SHARED((V2, D), jnp.float32),
          sh_c=pltpu.VMEM_SHARED((V2, D), jnp.float32),
          sem_a=pltpu.SemaphoreType.DMA((NBUF,)),
          sem_b=pltpu.SemaphoreType.DMA((NBUF,)),
          sem_c=pltpu.SemaphoreType.DMA((NBUF,)),
          sem_w=pltpu.SemaphoreType.DMA((NBUF,)),
      ),
  )
  def sc_kernel(ia_hbm, ib_hbm, ic_hbm, cd_hbm, ea_hbm, eb_hbm, ec_hbm,
                w_hbm, bias_hbm, out_hbm, *, idx_a, idx_b, idx_c, cont_v,
                wv, bv, buf_d, sh_b, sh_c, sem_a, sem_b, sem_c, sem_w):
    sid = lax.axis_index("s")
    wid = sid * NC + lax.axis_index("c")
    base = wid * (G * C)

    @pl.when(sid < CUT)
    def _():
      st = sid * R0
      pltpu.sync_copy(eb_hbm.at[pl.ds(st, R0)], sh_b.at[pl.ds(st, R0)])
      pltpu.sync_copy(ec_hbm.at[pl.ds(st, R0)], sh_c.at[pl.ds(st, R0)])

    @pl.when(sid >= CUT)
    def _():
      st = CUT * R0 + (sid - CUT) * R1
      pltpu.sync_copy(eb_hbm.at[pl.ds(st, R1)], sh_b.at[pl.ds(st, R1)])
      pltpu.sync_copy(ec_hbm.at[pl.ds(st, R1)], sh_c.at[pl.ds(st, R1)])

    pltpu.sync_copy(ia_hbm.at[wid], idx_a)
    pltpu.sync_copy(ib_hbm.at[wid], idx_b)
    pltpu.sync_copy(ic_hbm.at[wid], idx_c)
    pltpu.sync_copy(cd_hbm.at[wid], cont_v)
    pltpu.sync_copy(w_hbm, wv)
    pltpu.sync_copy(bias_hbm, bv)

    plsc.subcore_barrier()

    tabs = (ea_hbm, sh_b, sh_c)
    idxs = (idx_a, idx_b, idx_c)
    gsems = (sem_a, sem_b, sem_c)

    wk = [wv[pl.ds(k * LANES, LANES)] for k in range(grp)]
    bk = [bv[pl.ds(k * LANES, LANES)] for k in range(grp)]

    def g_copies(g, s):
      return [pltpu.make_async_copy(tabs[t].at[idxs[t].at[g]],
                                    buf_d.at[s], gsems[t].at[s])
              for t in range(3)]

    def w_copy(g, s):
      return pltpu.make_async_copy(
          buf_d.at[s], out_hbm.at[pl.ds(base + g * C, C)], sem_w.at[s])

    def compute_dense(g, s):
      def row16(j, carry2):
        cv = cont_v[g, pl.ds(j * LANES, LANES)]
        for r in range(LANES):
          c0 = cv[r]
          i = j * LANES + r
          for k in range(grp):
            buf_d[s, i, pl.ds(k * LANES, LANES)] = c0 * wk[k] + bk[k]
        return carry2

      lax.fori_loop(0, C // LANES, row16, 0, unroll=False)

    def body(g, carry):
      s = g % NBUF

      @pl.when(g >= NBUF)
      def _():
        w_copy(g - NBUF, s).wait()

      compute_dense(g, s)
      for t in range(3):
        pltpu.async_copy(tabs[t].at[idxs[t].at[g]], buf_d.at[s],
                         gsems[t].at[s], add=True)

      @pl.when(g >= 2)
      def _():
        sm2 = (g - 2) % NBUF
        for cp in g_copies(g - 2, sm2):
          cp.wait()
        w_copy(g - 2, sm2).start()

      return carry

    lax.fori_loop(0, G, body, 0, unroll=False)

    for gg in (G - 2, G - 1):
      sg = gg % NBUF
      for cp in g_copies(gg, sg):
        cp.wait()
      w_copy(gg, sg).start()
    for gg in range(G - NBUF, G):
      w_copy(gg, gg % NBUF).wait()

  return sc_kernel


def kernel(cat_a, cat_b, cat_c, cont_d, emb_a, emb_b, emb_c, W, b):
  B, S = cat_a.shape
  D = emb_a.shape[1]
  N = B * S
  per_w = N // NW
  G = per_w // C
  ia = cat_a.reshape(NW, G, C)
  ib = cat_b.reshape(NW, G, C)
  ic = cat_c.reshape(NW, G, C)
  cd = cont_d.reshape(NW, G, C)
  out = _make_sc_kernel(G, D, N)(
      ia, ib, ic, cd, emb_a, emb_b, emb_c, W.reshape(D), b)
  return out.reshape(B, S, D)
